# baseline (speedup 1.0000x reference)
import jax
import jax.numpy as jnp
from functools import partial
from jax.experimental import pallas as pl

_NUM_GRAPHS = 256
_CLUSTERS = [20, 5]


def _sage_block_kernel(h_ref, agg_ref, ws_ref, wn_ref, b_ref, o_ref, *, relu):
    out = jnp.dot(h_ref[...], ws_ref[...], preferred_element_type=jnp.float32)
    out = out + jnp.dot(agg_ref[...], wn_ref[...], preferred_element_type=jnp.float32)
    out = out + b_ref[...]
    if relu:
        out = jnp.maximum(out, 0.0)
    o_ref[...] = out


def _fused_sage_matmul(h, agg, Ws, Wn, b, relu):
    N, D = h.shape
    U = Ws.shape[1]
    BN = 400
    Np = ((N + BN - 1) // BN) * BN
    Up = max(U, 128)
    hp = jnp.pad(h, ((0, Np - N), (0, 0)))
    aggp = jnp.pad(agg, ((0, Np - N), (0, 0)))
    Wsp = jnp.pad(Ws, ((0, 0), (0, Up - U)))
    Wnp = jnp.pad(Wn, ((0, 0), (0, Up - U)))
    bp = jnp.pad(b, (0, Up - U)).reshape(1, Up)
    out = pl.pallas_call(
        partial(_sage_block_kernel, relu=relu),
        grid=(Np // BN,),
        in_specs=[
            pl.BlockSpec((BN, D), lambda i: (i, 0)),
            pl.BlockSpec((BN, D), lambda i: (i, 0)),
            pl.BlockSpec((D, Up), lambda i: (0, 0)),
            pl.BlockSpec((D, Up), lambda i: (0, 0)),
            pl.BlockSpec((1, Up), lambda i: (0, 0)),
        ],
        out_specs=pl.BlockSpec((BN, Up), lambda i: (i, 0)),
        out_shape=jax.ShapeDtypeStruct((Np, Up), jnp.float32),
    )(hp, aggp, Wsp, Wnp, bp)
    return out[:N, :U]


def _mean_sage(h, edge_index, edge_weight, W_self, W_neigh, b, relu):
    num_nodes = h.shape[0]
    src, dst = edge_index[0], edge_index[1]
    msgs = h[src] * edge_weight[:, None]
    agg = jax.ops.segment_sum(msgs, dst, num_segments=num_nodes)
    deg = jax.ops.segment_sum(edge_weight, dst, num_segments=num_nodes)
    agg = agg / jnp.maximum(deg[:, None], 1e-12)
    return _fused_sage_matmul(h, agg, W_self, W_neigh, b, relu)


def _gcn_model(h, edge_index, edge_weight, params):
    n = len(params)
    for i, (Ws, Wn, b) in enumerate(params):
        h = _mean_sage(h, edge_index, edge_weight, Ws, Wn, b, relu=(i < n - 1))
    return h


def _diffpool(h, edge_index, edge_weight, node_graph_index, num_graphs,
              feat_params, assign_params, W_units, b_units, num_clusters):
    z = _gcn_model(h, edge_index, edge_weight, feat_params)
    s_logits = _gcn_model(h, edge_index, edge_weight, assign_params)
    s = jax.nn.softmax(s_logits, axis=-1)

    C = num_clusters
    contrib = s[:, :, None] * z[:, None, :]
    pooled_h = jax.ops.segment_sum(contrib, node_graph_index, num_segments=num_graphs)
    pooled_h = pooled_h.reshape(num_graphs * C, -1)

    src, dst = edge_index[0], edge_index[1]
    e_graph = node_graph_index[src]
    e_contrib = edge_weight[:, None, None] * (s[src][:, :, None] * s[dst][:, None, :])
    pooled_adj = jax.ops.segment_sum(e_contrib, e_graph, num_segments=num_graphs)

    g_ids = jnp.arange(num_graphs)
    cc = jnp.arange(C)
    src_p = (g_ids[:, None, None] * C + cc[None, :, None]) + jnp.zeros((1, 1, C), dtype=jnp.int32)
    dst_p = (g_ids[:, None, None] * C + cc[None, None, :]) + jnp.zeros((1, C, 1), dtype=jnp.int32)
    pooled_edge_index = jnp.stack([src_p.reshape(-1), dst_p.reshape(-1)], axis=0)
    pooled_edge_weight = pooled_adj.reshape(-1)

    zero_w = jnp.zeros_like(W_units)
    pooled_h = _fused_sage_matmul(pooled_h, pooled_h, W_units, zero_w, b_units, relu=True)

    pooled_node_graph_index = jnp.repeat(jnp.arange(num_graphs), C)
    return pooled_h, pooled_edge_index, pooled_edge_weight, pooled_node_graph_index, s


def _min_cut_loss(edge_index, edge_weight, node_graph_index, s, num_graphs):
    src, dst = edge_index[0], edge_index[1]
    C = s.shape[1]
    e_graph = node_graph_index[src]
    cut_num_e = edge_weight * jnp.sum(s[src] * s[dst], axis=-1)
    cut_num = jax.ops.segment_sum(cut_num_e, e_graph, num_segments=num_graphs)
    deg = jax.ops.segment_sum(edge_weight, dst, num_segments=s.shape[0])
    deg_term = deg * jnp.sum(s * s, axis=-1)
    cut_den = jax.ops.segment_sum(deg_term, node_graph_index, num_segments=num_graphs)
    min_cut_losses = -(cut_num / jnp.maximum(cut_den, 1e-12))

    sts_contrib = s[:, :, None] * s[:, None, :]
    sts = jax.ops.segment_sum(sts_contrib, node_graph_index, num_segments=num_graphs)
    sts_norm = jnp.sqrt(jnp.maximum(jnp.sum(sts * sts, axis=(1, 2), keepdims=True), 1e-12))
    eye = jnp.eye(C)[None] / jnp.sqrt(C)
    diff = sts / sts_norm - eye
    orth_losses = jnp.sqrt(jnp.maximum(jnp.sum(diff * diff, axis=(1, 2)), 1e-12))
    return min_cut_losses, orth_losses


def kernel(x, edge_index, edge_weight, node_graph_index, feat0, assign0, Wu0, bu0,
           feat1, assign1, Wu1, bu1, W_mlp, b_mlp):
    h = x
    ei = edge_index.astype(jnp.int32)
    ew = edge_weight
    ngi = node_graph_index.astype(jnp.int32)
    num_graphs = _NUM_GRAPHS
    graph_h_list = []
    cut_loss = 0.0
    orth_loss = 0.0
    layers = [(feat0, assign0, Wu0, bu0, _CLUSTERS[0]),
              (feat1, assign1, Wu1, bu1, _CLUSTERS[1])]
    for fp, ap, Wu, bu, C in layers:
        ph, pei, pew, pngi, s = _diffpool(h, ei, ew, ngi, num_graphs, fp, ap, Wu, bu, C)
        gh = jax.ops.segment_max(ph, pngi, num_segments=num_graphs)
        graph_h_list.append(gh)
        mc, ol = _min_cut_loss(ei, ew, ngi, s, num_graphs)
        cut_loss = cut_loss + jnp.mean(mc)
        orth_loss = orth_loss + jnp.mean(ol)
        h, ei, ew, ngi = ph, pei, pew, pngi
    graph_h = jnp.concatenate(graph_h_list, axis=-1)
    logits = graph_h @ W_mlp + b_mlp
    return logits
